# baseline (speedup 1.0000x reference)
import functools

import jax
import jax.numpy as jnp
from jax.experimental import pallas as pl
from jax.experimental.pallas import tpu as pltpu

_N = 10000
_D = 128
_BI = 400


def _layer1_kernel(x_ref, w1t_ref, b1_ref, w2t_ref, a_ref, g_ref, y_ref):
    @pl.when(pl.program_id(0) == 0)
    def _():
        y_ref[...] = jnp.dot(x_ref[...], w1t_ref[...],
                             preferred_element_type=jnp.float32)

    h = jnp.dot(a_ref[...], y_ref[...], preferred_element_type=jnp.float32)
    h = jnp.maximum(h + b1_ref[...], 0.0)
    g_ref[...] = jnp.dot(h, w2t_ref[...], preferred_element_type=jnp.float32)


def _layer2_kernel(a_ref, g_ref, b2_ref, o_ref):
    o_ref[...] = jnp.dot(a_ref[...], g_ref[...],
                         preferred_element_type=jnp.float32) + b2_ref[...]


@functools.partial(jax.jit, static_argnames=())
def kernel(x, adj, W1, b1, W2, b2):
    n, d = adj.shape[0], x.shape[1]
    grid = (n // _BI,)
    b1r = b1.reshape(1, -1)
    b2r = b2.reshape(1, -1)

    g = pl.pallas_call(
        _layer1_kernel,
        grid=grid,
        in_specs=[
            pl.BlockSpec((n, d), lambda i: (0, 0)),
            pl.BlockSpec((d, d), lambda i: (0, 0)),
            pl.BlockSpec((1, d), lambda i: (0, 0)),
            pl.BlockSpec((d, d), lambda i: (0, 0)),
            pl.BlockSpec((_BI, n), lambda i: (i, 0)),
        ],
        out_specs=pl.BlockSpec((_BI, d), lambda i: (i, 0)),
        out_shape=jax.ShapeDtypeStruct((n, d), jnp.float32),
        scratch_shapes=[pltpu.VMEM((n, d), jnp.float32)],
        compiler_params=pltpu.CompilerParams(
            dimension_semantics=("arbitrary",),
        ),
    )(x, W1.T, b1r, W2.T, adj)

    out = pl.pallas_call(
        _layer2_kernel,
        grid=grid,
        in_specs=[
            pl.BlockSpec((_BI, n), lambda i: (i, 0)),
            pl.BlockSpec((n, d), lambda i: (0, 0)),
            pl.BlockSpec((1, d), lambda i: (0, 0)),
        ],
        out_specs=pl.BlockSpec((_BI, d), lambda i: (i, 0)),
        out_shape=jax.ShapeDtypeStruct((n, d), jnp.float32),
        compiler_params=pltpu.CompilerParams(
            dimension_semantics=("arbitrary",),
        ),
    )(adj, g, b2r)

    return out


# speedup vs baseline: 1.0302x; 1.0302x over previous
import functools

import jax
import jax.numpy as jnp
from jax.experimental import pallas as pl
from jax.experimental.pallas import tpu as pltpu

_N = 10000
_D = 128
_BI = 400
_NB = _N // _BI


def _gcn_kernel(x_ref, w1t_ref, b1_ref, w2t_ref, b2_ref, a_ref,
                o_ref, y_ref, g_ref):
    i = pl.program_id(0)

    @pl.when(i == 0)
    def _():
        y_ref[...] = jnp.dot(x_ref[...], w1t_ref[...],
                             preferred_element_type=jnp.float32)

    @pl.when(i < _NB)
    def _():
        h = jnp.dot(a_ref[...], y_ref[...],
                    preferred_element_type=jnp.float32)
        h = jnp.maximum(h + b1_ref[...], 0.0)
        g_ref[pl.ds(i * _BI, _BI), :] = jnp.dot(
            h, w2t_ref[...], preferred_element_type=jnp.float32)

    @pl.when(i >= _NB)
    def _():
        o_ref[...] = jnp.dot(a_ref[...], g_ref[...],
                             preferred_element_type=jnp.float32) + b2_ref[...]


@functools.partial(jax.jit, static_argnames=())
def kernel(x, adj, W1, b1, W2, b2):
    n, d = adj.shape[0], x.shape[1]
    nb = n // _BI
    b1r = b1.reshape(1, -1)
    b2r = b2.reshape(1, -1)

    out = pl.pallas_call(
        _gcn_kernel,
        grid=(2 * nb,),
        in_specs=[
            pl.BlockSpec((n, d), lambda i: (0, 0)),
            pl.BlockSpec((d, d), lambda i: (0, 0)),
            pl.BlockSpec((1, d), lambda i: (0, 0)),
            pl.BlockSpec((d, d), lambda i: (0, 0)),
            pl.BlockSpec((1, d), lambda i: (0, 0)),
            pl.BlockSpec((_BI, n), lambda i: (i % nb, 0)),
        ],
        out_specs=pl.BlockSpec((_BI, d),
                               lambda i: (jnp.maximum(i - nb, 0), 0)),
        out_shape=jax.ShapeDtypeStruct((n, d), jnp.float32),
        scratch_shapes=[
            pltpu.VMEM((n, d), jnp.float32),
            pltpu.VMEM((n, d), jnp.float32),
        ],
        compiler_params=pltpu.CompilerParams(
            dimension_semantics=("arbitrary",),
        ),
    )(x, W1.T, b1r, W2.T, b2r, adj)

    return out
